# Initial kernel scaffold; baseline (speedup 1.0000x reference)
import jax
import jax.numpy as jnp
from jax.experimental import pallas as pl


def kernel(x, edge_index, W1, b1, W2, b2):
    raise NotImplementedError("write your pallas kernel here")


# speedup vs baseline: 13.1040x; 13.1040x over previous
import functools

import jax
import jax.numpy as jnp
from jax import lax
from jax.experimental import pallas as pl
from jax.experimental.pallas import tpu as pltpu
from jax.experimental.pallas import tpu_sc as plsc

N = 10000
E = 320000
D_IN = 128
D_HID = 128
D_OUT = 64

NC, NS = 2, 16
EB = 128
NB_W = 79
E_PAD = NC * NS * NB_W * EB
N_ACC = 10240
ROWS_SUB = N_ACC // NS
PAD_ROW = N_ACC - 8

_sc_mesh = plsc.VectorSubcoreMesh(core_axis_name="c", subcore_axis_name="s")


def _hist_body(dst_hbm, out_hbm, dst_v, ones_v, zro_v, acc_sh):
    c = lax.axis_index("c")
    s = lax.axis_index("s")
    pltpu.sync_copy(dst_hbm.at[c, s], dst_v)

    @pl.loop(0, EB)
    def _(i):
        ones_v.at[pl.ds(i, 1), pl.ds(0, 16)][...] = jnp.ones((1, 16), jnp.float32)
        zro_v.at[pl.ds(i, 1), pl.ds(0, 16)][...] = jnp.zeros((1, 16), jnp.float32)

    @pl.loop(0, ROWS_SUB // EB)
    def _(k):
        pltpu.sync_copy(zro_v, acc_sh.at[pl.ds(s * ROWS_SUB + k * EB, EB)])

    plsc.subcore_barrier()

    @pl.loop(0, NB_W)
    def _(b):
        pltpu.sync_copy(ones_v, acc_sh.at[dst_v.at[b]], add=True)

    plsc.subcore_barrier()

    @pl.loop(0, ROWS_SUB // EB)
    def _(k):
        off = s * ROWS_SUB + k * EB
        pltpu.sync_copy(acc_sh.at[pl.ds(off, EB)], out_hbm.at[c, pl.ds(off, EB)])


_hist = functools.partial(
    pl.kernel,
    out_type=jax.ShapeDtypeStruct((NC, N_ACC, 16), jnp.float32),
    mesh=_sc_mesh,
    scratch_types=[
        pltpu.VMEM((NB_W, EB), jnp.int32),
        pltpu.VMEM((EB, 16), jnp.float32),
        pltpu.VMEM((EB, 16), jnp.float32),
        pltpu.VMEM_SHARED((N_ACC, 16), jnp.float32),
    ],
)(_hist_body)


def _msg_body(D, table_hbm, src_hbm, dst_hbm, out_hbm,
              src_v, dst_v, rows_v, acc_sh, sem):
    c = lax.axis_index("c")
    s = lax.axis_index("s")
    pltpu.sync_copy(src_hbm.at[c, s], src_v)
    pltpu.sync_copy(dst_hbm.at[c, s], dst_v)

    @pl.loop(0, EB)
    def _(i):
        @pl.loop(0, D, step=16)
        def _(j):
            rows_v.at[pl.ds(i, 1), pl.ds(j, 16)][...] = jnp.zeros((1, 16), jnp.float32)

    @pl.loop(0, ROWS_SUB // EB)
    def _(k):
        pltpu.sync_copy(rows_v, acc_sh.at[pl.ds(s * ROWS_SUB + k * EB, EB)])

    plsc.subcore_barrier()

    @pl.loop(0, NB_W)
    def _(b):
        pltpu.async_copy(table_hbm.at[src_v.at[b]], rows_v, sem).wait()
        pltpu.sync_copy(rows_v, acc_sh.at[dst_v.at[b]], add=True)

    plsc.subcore_barrier()

    @pl.loop(0, ROWS_SUB // EB)
    def _(k):
        off = s * ROWS_SUB + k * EB
        pltpu.sync_copy(acc_sh.at[pl.ds(off, EB)], out_hbm.at[c, pl.ds(off, EB)])


def _make_msg(D):
    return functools.partial(
        pl.kernel,
        out_type=jax.ShapeDtypeStruct((NC, N_ACC, D), jnp.float32),
        mesh=_sc_mesh,
        scratch_types=[
            pltpu.VMEM((NB_W, EB), jnp.int32),
            pltpu.VMEM((NB_W, EB), jnp.int32),
            pltpu.VMEM((EB, D), jnp.float32),
            pltpu.VMEM_SHARED((N_ACC, D), jnp.float32),
            pltpu.SemaphoreType.DMA,
        ],
    )(functools.partial(_msg_body, D))


_msg128 = _make_msg(D_HID)


_BLK = 1000


def _mm1_kern(x_ref, w_ref, o_ref):
    o_ref[...] = jnp.dot(x_ref[...], w_ref[...],
                         preferred_element_type=jnp.float32,
                         precision=lax.Precision.HIGHEST)


def _mm1(x, W1):
    return pl.pallas_call(
        _mm1_kern,
        grid=(N // _BLK,),
        in_specs=[pl.BlockSpec((_BLK, D_IN), lambda i: (i, 0)),
                  pl.BlockSpec((D_IN, D_HID), lambda i: (0, 0))],
        out_specs=pl.BlockSpec((_BLK, D_HID), lambda i: (i, 0)),
        out_shape=jax.ShapeDtypeStruct((N, D_HID), jnp.float32),
    )(x, W1)


def _dinv_scale_kern(d0_ref, d1_ref, h1_ref, dinv_ref, h1p_ref):
    deg = d0_ref[...][:, :1] + d1_ref[...][:, :1] + 1.0
    dinv = lax.rsqrt(deg)
    dinv_ref[...] = dinv
    h1p_ref[...] = dinv * h1_ref[...]


def _dinv_scale(d0, d1, h1):
    return pl.pallas_call(
        _dinv_scale_kern,
        grid=(N // _BLK,),
        in_specs=[pl.BlockSpec((_BLK, 16), lambda i: (i, 0)),
                  pl.BlockSpec((_BLK, 16), lambda i: (i, 0)),
                  pl.BlockSpec((_BLK, D_HID), lambda i: (i, 0))],
        out_specs=[pl.BlockSpec((_BLK, 1), lambda i: (i, 0)),
                   pl.BlockSpec((_BLK, D_HID), lambda i: (i, 0))],
        out_shape=[jax.ShapeDtypeStruct((N, 1), jnp.float32),
                   jax.ShapeDtypeStruct((N, D_HID), jnp.float32)],
    )(d0, d1, h1)


def _layer1_kern(pa_ref, pb_ref, h1_ref, dinv_ref, b1_ref, w2_ref,
                 h2_ref, h2p_ref):
    dinv = dinv_ref[...]
    h = dinv * (pa_ref[...] + pb_ref[...]) + (dinv * dinv) * h1_ref[...] + b1_ref[...]
    h = jnp.maximum(h, 0.0)
    h2 = jnp.dot(h, w2_ref[...], preferred_element_type=jnp.float32,
                 precision=lax.Precision.HIGHEST)
    h2_ref[...] = h2
    h2p_ref[...] = jnp.concatenate(
        [dinv * h2, jnp.zeros((h2.shape[0], D_HID - D_OUT), jnp.float32)], axis=1)


def _layer1_finish(pa, pb, h1, dinv, b1, W2):
    return pl.pallas_call(
        _layer1_kern,
        grid=(N // _BLK,),
        in_specs=[pl.BlockSpec((_BLK, D_HID), lambda i: (i, 0)),
                  pl.BlockSpec((_BLK, D_HID), lambda i: (i, 0)),
                  pl.BlockSpec((_BLK, D_HID), lambda i: (i, 0)),
                  pl.BlockSpec((_BLK, 1), lambda i: (i, 0)),
                  pl.BlockSpec((1, D_HID), lambda i: (0, 0)),
                  pl.BlockSpec((D_HID, D_OUT), lambda i: (0, 0))],
        out_specs=[pl.BlockSpec((_BLK, D_OUT), lambda i: (i, 0)),
                   pl.BlockSpec((_BLK, D_HID), lambda i: (i, 0))],
        out_shape=[jax.ShapeDtypeStruct((N, D_OUT), jnp.float32),
                   jax.ShapeDtypeStruct((N, D_HID), jnp.float32)],
    )(pa, pb, h1, dinv, b1, W2)


def _final_kern(pa_ref, pb_ref, h2_ref, dinv_ref, b2_ref, z_ref):
    dinv = dinv_ref[...]
    z_ref[...] = (dinv * (pa_ref[...] + pb_ref[...])
                  + (dinv * dinv) * h2_ref[...] + b2_ref[...])


def _final(pa, pb, h2, dinv, b2):
    return pl.pallas_call(
        _final_kern,
        grid=(N // _BLK,),
        in_specs=[pl.BlockSpec((_BLK, D_OUT), lambda i: (i, 0)),
                  pl.BlockSpec((_BLK, D_OUT), lambda i: (i, 0)),
                  pl.BlockSpec((_BLK, D_OUT), lambda i: (i, 0)),
                  pl.BlockSpec((_BLK, 1), lambda i: (i, 0)),
                  pl.BlockSpec((1, D_OUT), lambda i: (0, 0))],
        out_specs=pl.BlockSpec((_BLK, D_OUT), lambda i: (i, 0)),
        out_shape=jax.ShapeDtypeStruct((N, D_OUT), jnp.float32),
    )(pa, pb, h2, dinv, b2)


def kernel(x, edge_index, W1, b1, W2, b2):
    src = edge_index[0].astype(jnp.int32)
    dst = edge_index[1].astype(jnp.int32)
    pad = E_PAD - E
    src_r = jnp.concatenate([src, jnp.zeros((pad,), jnp.int32)]).reshape(NC, NS, NB_W, EB)
    dst_r = jnp.concatenate([dst, jnp.full((pad,), PAD_ROW, jnp.int32)]).reshape(NC, NS, NB_W, EB)

    degp = _hist(dst_r)
    h1 = _mm1(x, W1)
    dinv, h1p = _dinv_scale(degp[0, :N], degp[1, :N], h1)
    acc1 = _msg128(h1p, src_r, dst_r)
    h2, h2p = _layer1_finish(acc1[0, :N], acc1[1, :N], h1, dinv,
                             b1.reshape(1, -1), W2)
    acc2 = _msg128(h2p, src_r, dst_r)
    return _final(acc2[0, :N, :D_OUT], acc2[1, :N, :D_OUT], h2, dinv,
                  b2.reshape(1, -1))
